# baseline (speedup 1.0000x reference)
import functools

import jax
import jax.numpy as jnp
from jax import lax
from jax.experimental import pallas as pl
from jax.experimental.pallas import tpu as pltpu
from jax.experimental.pallas import tpu_sc as plsc

F = 26
V = 100000
E = 32
B = 16384
EPS = 1e-5

NC, NS, L = 2, 16, 16
NW = NC * NS
CH0 = 50048
CH1 = 49920
TAILV = CH0 + CH1
TAILW = 128
OROWS = 32


def _sc_gather(xT, tables_t, tables_tail, f0, nf):
    mesh = plsc.VectorSubcoreMesh(
        core_axis_name="c", subcore_axis_name="s", num_cores=NC, num_subcores=NS
    )

    @functools.partial(
        pl.kernel,
        out_type=jax.ShapeDtypeStruct((nf * E, 128, 128), jnp.float32),
        mesh=mesh,
        scratch_types=[
            pltpu.VMEM((TAILV + TAILW,), jnp.float32),
            pltpu.VMEM((B,), jnp.int32),
            pltpu.VMEM((2, OROWS, 128), jnp.float32),
            pltpu.SemaphoreType.DMA,
            pltpu.SemaphoreType.DMA,
            pltpu.SemaphoreType.DMA,
            pltpu.SemaphoreType.DMA,
        ],
        compiler_params=pltpu.CompilerParams(
            use_tc_tiling_on_sc=True, needs_layout_passes=False
        ),
    )
    def gather_kernel(x_hbm, tab_hbm, tail_hbm, out_hbm, vec_v, idx_v, out_v,
                      vsem, isem, osem0, osem1):
        e = lax.axis_index("s") * NC + lax.axis_index("c")
        osems = (osem0, osem1)
        ocps = [None, None]
        nq = B // (OROWS * 128)

        for j in range(nf):
            f = f0 + j
            vcps = [
                pltpu.async_copy(tab_hbm.at[f, e, pl.ds(0, CH0)],
                                 vec_v.at[pl.ds(0, CH0)], vsem),
                pltpu.async_copy(tab_hbm.at[f, e, pl.ds(CH0, CH1)],
                                 vec_v.at[pl.ds(CH0, CH1)], vsem),
                pltpu.async_copy(tail_hbm.at[f, e],
                                 vec_v.at[pl.ds(TAILV, TAILW)], vsem),
            ]
            icp = pltpu.async_copy(x_hbm.at[f], idx_v, isem)
            for cp in vcps:
                cp.wait()
            icp.wait()
            t = j * E + e

            for q in range(nq):
                p = q % 2
                if ocps[p] is not None:
                    ocps[p].wait()
                base = q * (OROWS * 128)

                def row(r, c2, base=base, p=p):
                    pos = base + r * 128
                    for u in range(128 // L):
                        idx = idx_v[pl.ds(pos + u * L, L)]
                        out_v[p, r, pl.ds(u * L, L)] = plsc.load_gather(
                            vec_v, [idx]
                        )
                    return c2

                lax.fori_loop(0, OROWS, row, 0)
                ocps[p] = pltpu.async_copy(
                    out_v.at[p], out_hbm.at[t, pl.ds(q * OROWS, OROWS), :],
                    osems[p],
                )
        ocps[0].wait()
        ocps[1].wait()

    return gather_kernel(xT, tables_t, tables_tail)


BB = 128
KSUB = 8
NBLK = B // (BB * KSUB)
FH = F // 2
HD = FH * E


def _ln_relu_t(h, g, b):
    m = jnp.mean(h, axis=0, keepdims=True)
    v = jnp.mean((h - m) ** 2, axis=0, keepdims=True)
    return jnp.maximum((h - m) / jnp.sqrt(v + EPS) * g + b, 0.0)


def _mlp_part_body(emb_ref, w1t_ref, out_ref):
    for k in range(KSUB):
        out_ref[0, :, pl.ds(k * BB, BB)] = jnp.dot(
            w1t_ref[...], emb_ref[:, k, :], preferred_element_type=jnp.float32
        )


def _tc_mlp_part(emb3, w1t):
    full = lambda s: pl.BlockSpec(s, lambda i: (0,) * len(s))
    return pl.pallas_call(
        _mlp_part_body,
        grid=(NBLK,),
        in_specs=[pl.BlockSpec((HD, KSUB, BB), lambda i: (0, i, 0)),
                  full((64, HD))],
        out_specs=pl.BlockSpec((1, 64, KSUB * BB), lambda i: (i, 0, 0)),
        out_shape=jax.ShapeDtypeStruct((NBLK, 64, KSUB * BB), jnp.float32),
    )(emb3, w1t)


def _mlp_body(emb_ref, part_ref, w1t_ref, b1_ref, g1_ref, be1_ref, w2t_ref,
              b2_ref, g2_ref, be2_ref, w3t_ref, b3_ref, g3_ref, be3_ref,
              wf_ref, bf_ref, out_ref):
    for k in range(KSUB):
        eb = emb_ref[:, k, :]
        h1 = (jnp.dot(w1t_ref[...], eb, preferred_element_type=jnp.float32)
              + part_ref[0, :, pl.ds(k * BB, BB)] + b1_ref[...])
        h = _ln_relu_t(h1, g1_ref[...], be1_ref[...])
        h = _ln_relu_t(
            jnp.dot(w2t_ref[...], h, preferred_element_type=jnp.float32)
            + b2_ref[...], g2_ref[...], be2_ref[...])
        h = _ln_relu_t(
            jnp.dot(w3t_ref[...], h, preferred_element_type=jnp.float32)
            + b3_ref[...], g3_ref[...], be3_ref[...])
        logits = jnp.sum(h * wf_ref[...], axis=0) + bf_ref[0, 0]
        out_ref[0, k, :] = 1.0 / (1.0 + jnp.exp(-logits))


def _tc_mlp(emb3, part, W1t_b, b1, g1, be1, W2, b2, g2, be2, W3, b3, g3, be3,
            Wf, bf):
    col = lambda a: a.reshape(-1, 1)
    full = lambda s: pl.BlockSpec(s, lambda i: (0,) * len(s))
    out = pl.pallas_call(
        _mlp_body,
        grid=(NBLK,),
        in_specs=[
            pl.BlockSpec((HD, KSUB, BB), lambda i: (0, i, 0)),
            pl.BlockSpec((1, 64, KSUB * BB), lambda i: (i, 0, 0)),
            full((64, HD)),
            full((64, 1)), full((64, 1)), full((64, 1)),
            full((32, 64)), full((32, 1)), full((32, 1)), full((32, 1)),
            full((16, 32)), full((16, 1)), full((16, 1)), full((16, 1)),
            full((16, 1)), full((1, 1)),
        ],
        out_specs=pl.BlockSpec((1, KSUB, BB), lambda i: (i, 0, 0)),
        out_shape=jax.ShapeDtypeStruct((NBLK, KSUB, BB), jnp.float32),
    )(emb3, part, W1t_b, col(b1), col(g1), col(be1), W2.T, col(b2), col(g2),
      col(be2), W3.T, col(b3), col(g3), col(be3), Wf, bf.reshape(1, 1))
    return out.reshape(B)


def kernel(x, tables, W1, b1, g1, be1, W2, b2, g2, be2, W3, b3, g3, be3, Wf, bf):
    xT = x.T
    tables_t = tables.transpose(0, 2, 1)
    tables_tail = jnp.pad(tables_t[:, :, TAILV:],
                          ((0, 0), (0, 0), (0, TAILW - (V - TAILV))))
    embA = _sc_gather(xT, tables_t, tables_tail, 0, FH)
    embB = _sc_gather(xT, tables_t, tables_tail, FH, F - FH)
    part = _tc_mlp_part(embA, W1[:HD].T)
    return _tc_mlp(embB, part, W1[HD:].T, b1, g1, be1, W2, b2, g2, be2,
                   W3, b3, g3, be3, Wf, bf)


# speedup vs baseline: 1.1143x; 1.1143x over previous
import functools

import jax
import jax.numpy as jnp
from jax import lax
from jax.experimental import pallas as pl
from jax.experimental.pallas import tpu as pltpu
from jax.experimental.pallas import tpu_sc as plsc

F = 26
V = 100000
E = 32
B = 16384
EPS = 1e-5

NC, NS, L = 2, 16, 16
NW = NC * NS
CH0 = 50048
CH1 = 49920
TAILV = CH0 + CH1
TAILW = 128
OROWS = 32


def _sc_gather(xT, tables_t, tables_tail):
    mesh = plsc.VectorSubcoreMesh(
        core_axis_name="c", subcore_axis_name="s", num_cores=NC, num_subcores=NS
    )

    @functools.partial(
        pl.kernel,
        out_type=jax.ShapeDtypeStruct((F * E, 128, 128), jnp.float32),
        mesh=mesh,
        scratch_types=[
            pltpu.VMEM((TAILV + TAILW,), jnp.float32),
            pltpu.VMEM((B,), jnp.int32),
            pltpu.VMEM((2, OROWS, 128), jnp.float32),
            pltpu.SemaphoreType.DMA,
            pltpu.SemaphoreType.DMA,
            pltpu.SemaphoreType.DMA,
            pltpu.SemaphoreType.DMA,
        ],
        compiler_params=pltpu.CompilerParams(
            use_tc_tiling_on_sc=True, needs_layout_passes=False
        ),
    )
    def gather_kernel(x_hbm, tab_hbm, tail_hbm, out_hbm, vec_v, idx_v, out_v,
                      vsem, isem, osem0, osem1):
        e = lax.axis_index("s") * NC + lax.axis_index("c")
        osems = (osem0, osem1)
        ocps = [None, None]
        nq = B // (OROWS * 128)

        for f in range(F):
            vcps = [
                pltpu.async_copy(tab_hbm.at[f, e, pl.ds(0, CH0)],
                                 vec_v.at[pl.ds(0, CH0)], vsem),
                pltpu.async_copy(tab_hbm.at[f, e, pl.ds(CH0, CH1)],
                                 vec_v.at[pl.ds(CH0, CH1)], vsem),
                pltpu.async_copy(tail_hbm.at[f, e],
                                 vec_v.at[pl.ds(TAILV, TAILW)], vsem),
            ]
            icp = pltpu.async_copy(x_hbm.at[f], idx_v, isem)
            for cp in vcps:
                cp.wait()
            icp.wait()
            t = f * E + e

            for q in range(nq):
                p = q % 2
                if ocps[p] is not None:
                    ocps[p].wait()
                base = q * (OROWS * 128)

                def row(r, c2, base=base, p=p):
                    pos = base + r * 128
                    for u in range(128 // L):
                        idx = idx_v[pl.ds(pos + u * L, L)]
                        out_v[p, r, pl.ds(u * L, L)] = plsc.load_gather(
                            vec_v, [idx]
                        )
                    return c2

                lax.fori_loop(0, OROWS, row, 0)
                ocps[p] = pltpu.async_copy(
                    out_v.at[p], out_hbm.at[t, pl.ds(q * OROWS, OROWS), :],
                    osems[p],
                )
        ocps[0].wait()
        ocps[1].wait()

    return gather_kernel(xT, tables_t, tables_tail)


BB = 128
KSUB = 8
NBLK = B // (BB * KSUB)
D_IN = F * E


def _ln_relu_t(h, g, b):
    m = jnp.mean(h, axis=0, keepdims=True)
    v = jnp.mean((h - m) ** 2, axis=0, keepdims=True)
    return jnp.maximum((h - m) / jnp.sqrt(v + EPS) * g + b, 0.0)


def _mlp_body(emb_ref, w1t_ref, b1_ref, g1_ref, be1_ref, w2t_ref, b2_ref,
              g2_ref, be2_ref, w3t_ref, b3_ref, g3_ref, be3_ref, wf_ref,
              bf_ref, out_ref):
    h1 = jnp.concatenate(
        [jnp.dot(w1t_ref[...], emb_ref[:, k, :],
                 preferred_element_type=jnp.float32) for k in range(KSUB)],
        axis=1,
    )
    h = _ln_relu_t(h1 + b1_ref[...], g1_ref[...], be1_ref[...])
    h = _ln_relu_t(
        jnp.dot(w2t_ref[...], h, preferred_element_type=jnp.float32)
        + b2_ref[...], g2_ref[...], be2_ref[...])
    h = _ln_relu_t(
        jnp.dot(w3t_ref[...], h, preferred_element_type=jnp.float32)
        + b3_ref[...], g3_ref[...], be3_ref[...])
    logits = jnp.sum(h * wf_ref[...], axis=0, keepdims=True) + bf_ref[0, 0]
    out_ref[0] = 1.0 / (1.0 + jnp.exp(-logits))


def _tc_mlp(emb3, W1, b1, g1, be1, W2, b2, g2, be2, W3, b3, g3, be3, Wf, bf):
    col = lambda a: a.reshape(-1, 1)
    full = lambda s: pl.BlockSpec(s, lambda i: (0,) * len(s))
    out = pl.pallas_call(
        _mlp_body,
        grid=(NBLK,),
        in_specs=[
            pl.BlockSpec((D_IN, KSUB, BB), lambda i: (0, i, 0)),
            full((64, D_IN)),
            full((64, 1)), full((64, 1)), full((64, 1)),
            full((32, 64)), full((32, 1)), full((32, 1)), full((32, 1)),
            full((16, 32)), full((16, 1)), full((16, 1)), full((16, 1)),
            full((16, 1)), full((1, 1)),
        ],
        out_specs=pl.BlockSpec((1, 1, KSUB * BB), lambda i: (i, 0, 0)),
        out_shape=jax.ShapeDtypeStruct((NBLK, 1, KSUB * BB), jnp.float32),
    )(emb3, W1.T, col(b1), col(g1), col(be1), W2.T, col(b2), col(g2), col(be2),
      W3.T, col(b3), col(g3), col(be3), Wf, bf.reshape(1, 1))
    return out.reshape(B)


def kernel(x, tables, W1, b1, g1, be1, W2, b2, g2, be2, W3, b3, g3, be3, Wf, bf):
    xT = x.T
    tables_t = tables.transpose(0, 2, 1)
    tables_tail = jnp.pad(tables_t[:, :, TAILV:],
                          ((0, 0), (0, 0), (0, TAILW - (V - TAILV))))
    emb3 = _sc_gather(xT, tables_t, tables_tail)
    return _tc_mlp(emb3, W1, b1, g1, be1, W2, b2, g2, be2, W3, b3, g3, be3,
                   Wf, bf)


# speedup vs baseline: 1.1234x; 1.0081x over previous
import functools

import jax
import jax.numpy as jnp
from jax import lax
from jax.experimental import pallas as pl
from jax.experimental.pallas import tpu as pltpu
from jax.experimental.pallas import tpu_sc as plsc

F = 26
V = 100000
E = 32
B = 16384
EPS = 1e-5

NC, NS, L = 2, 16, 16
NW = NC * NS
CH0 = 50048
CH1 = 49920
TAILV = CH0 + CH1
TAILW = 128
OROWS = 32


def _sc_gather(xT, tables_t, tables_tail):
    mesh = plsc.VectorSubcoreMesh(
        core_axis_name="c", subcore_axis_name="s", num_cores=NC, num_subcores=NS
    )

    @functools.partial(
        pl.kernel,
        out_type=jax.ShapeDtypeStruct((F * E, 128, 128), jnp.float32),
        mesh=mesh,
        scratch_types=[
            pltpu.VMEM((TAILV + TAILW,), jnp.float32),
            pltpu.VMEM((B,), jnp.int32),
            pltpu.VMEM((2, OROWS, 128), jnp.float32),
            pltpu.SemaphoreType.DMA,
            pltpu.SemaphoreType.DMA,
            pltpu.SemaphoreType.DMA,
            pltpu.SemaphoreType.DMA,
        ],
        compiler_params=pltpu.CompilerParams(
            use_tc_tiling_on_sc=True, needs_layout_passes=False
        ),
    )
    def gather_kernel(x_hbm, tab_hbm, tail_hbm, out_hbm, vec_v, idx_v, out_v,
                      vsem, isem, osem0, osem1):
        e = lax.axis_index("s") * NC + lax.axis_index("c")
        osems = (osem0, osem1)
        ocps = [None, None]
        nq = B // (OROWS * 128)

        for f in range(F):
            vcps = [
                pltpu.async_copy(tab_hbm.at[f, e, pl.ds(0, CH0)],
                                 vec_v.at[pl.ds(0, CH0)], vsem),
                pltpu.async_copy(tab_hbm.at[f, e, pl.ds(CH0, CH1)],
                                 vec_v.at[pl.ds(CH0, CH1)], vsem),
                pltpu.async_copy(tail_hbm.at[f, e],
                                 vec_v.at[pl.ds(TAILV, TAILW)], vsem),
            ]
            icp = pltpu.async_copy(x_hbm.at[f], idx_v, isem)
            for cp in vcps:
                cp.wait()
            icp.wait()
            t = f * E + e

            for q in range(nq):
                p = q % 2
                if ocps[p] is not None:
                    ocps[p].wait()
                base = q * (OROWS * 128)

                def row(r, c2, base=base, p=p):
                    pos = base + r * 128
                    for u in range(128 // L):
                        idx = idx_v[pl.ds(pos + u * L, L)]
                        out_v[p, r, pl.ds(u * L, L)] = plsc.load_gather(
                            vec_v, [idx]
                        )
                    return c2

                lax.fori_loop(0, OROWS, row, 0)
                ocps[p] = pltpu.async_copy(
                    out_v.at[p], out_hbm.at[t, pl.ds(q * OROWS, OROWS), :],
                    osems[p],
                )
        ocps[0].wait()
        ocps[1].wait()

    return gather_kernel(xT, tables_t, tables_tail)


BB = 128
KSUB = 16
NBLK = B // (BB * KSUB)
D_IN = F * E


def _ln_relu_t(h, g, b):
    m = jnp.mean(h, axis=0, keepdims=True)
    v = jnp.mean((h - m) ** 2, axis=0, keepdims=True)
    return jnp.maximum((h - m) / jnp.sqrt(v + EPS) * g + b, 0.0)


def _mlp_body(emb_ref, w1t_ref, b1_ref, g1_ref, be1_ref, w2t_ref, b2_ref,
              g2_ref, be2_ref, w3t_ref, b3_ref, g3_ref, be3_ref, wf_ref,
              bf_ref, out_ref):
    h1 = jnp.concatenate(
        [jnp.dot(w1t_ref[...], emb_ref[:, k, :],
                 preferred_element_type=jnp.float32) for k in range(KSUB)],
        axis=1,
    )
    h = _ln_relu_t(h1 + b1_ref[...], g1_ref[...], be1_ref[...])
    h = _ln_relu_t(
        jnp.dot(w2t_ref[...], h, preferred_element_type=jnp.float32)
        + b2_ref[...], g2_ref[...], be2_ref[...])
    h = _ln_relu_t(
        jnp.dot(w3t_ref[...], h, preferred_element_type=jnp.float32)
        + b3_ref[...], g3_ref[...], be3_ref[...])
    logits = jnp.sum(h * wf_ref[...], axis=0, keepdims=True) + bf_ref[0, 0]
    out_ref[0] = 1.0 / (1.0 + jnp.exp(-logits))


def _tc_mlp(emb3, W1, b1, g1, be1, W2, b2, g2, be2, W3, b3, g3, be3, Wf, bf):
    col = lambda a: a.reshape(-1, 1)
    full = lambda s: pl.BlockSpec(s, lambda i: (0,) * len(s))
    out = pl.pallas_call(
        _mlp_body,
        grid=(NBLK,),
        in_specs=[
            pl.BlockSpec((D_IN, KSUB, BB), lambda i: (0, i, 0)),
            full((64, D_IN)),
            full((64, 1)), full((64, 1)), full((64, 1)),
            full((32, 64)), full((32, 1)), full((32, 1)), full((32, 1)),
            full((16, 32)), full((16, 1)), full((16, 1)), full((16, 1)),
            full((16, 1)), full((1, 1)),
        ],
        out_specs=pl.BlockSpec((1, 1, KSUB * BB), lambda i: (i, 0, 0)),
        out_shape=jax.ShapeDtypeStruct((NBLK, 1, KSUB * BB), jnp.float32),
    )(emb3, W1.T, col(b1), col(g1), col(be1), W2.T, col(b2), col(g2), col(be2),
      W3.T, col(b3), col(g3), col(be3), Wf, bf.reshape(1, 1))
    return out.reshape(B)


def kernel(x, tables, W1, b1, g1, be1, W2, b2, g2, be2, W3, b3, g3, be3, Wf, bf):
    xT = x.T
    tables_t = tables.transpose(0, 2, 1)
    tables_tail = jnp.pad(tables_t[:, :, TAILV:],
                          ((0, 0), (0, 0), (0, TAILW - (V - TAILV))))
    emb3 = _sc_gather(xT, tables_t, tables_tail)
    return _tc_mlp(emb3, W1, b1, g1, be1, W2, b2, g2, be2, W3, b3, g3, be3,
                   Wf, bf)
